# baseline (speedup 1.0000x reference)
import functools

import jax
import jax.numpy as jnp
from jax import lax
from jax.experimental import pallas as pl
from jax.experimental.pallas import tpu as pltpu
from jax.experimental.pallas import tpu_sc as plsc

D_MODEL = 768
N_EXPERTS = 64
TOP_K = 2
D_FF = 4 * D_MODEL
T_TOKENS = 2048
N_ASSIGN = T_TOKENS * TOP_K

BT = 128
NBLK = N_ASSIGN // BT + N_EXPERTS - 1
NP = NBLK * BT
NBLK_PAD = 128


def _route_body(x_ref, wg_ref, bgb_ref,
                pos1_ref, pos2_ref, blke_ref, w1b_ref, w2b_ref):
    x = x_ref[...]
    logits = x @ wg_ref[...] + bgb_ref[...]
    s = jax.nn.sigmoid(logits)
    lane = jax.lax.broadcasted_iota(jnp.int32, s.shape, 1)
    big = jnp.int32(N_EXPERTS)
    m1 = jnp.max(s, axis=1, keepdims=True)
    i1 = jnp.min(jnp.where(s == m1, lane, big), axis=1, keepdims=True)
    s2 = jnp.where(lane == i1, -jnp.inf, s)
    m2 = jnp.max(s2, axis=1, keepdims=True)
    i2 = jnp.min(jnp.where(s2 == m2, lane, big), axis=1, keepdims=True)
    ones16 = jnp.ones((1, 16), jnp.float32)
    w1b_ref[...] = jax.nn.sigmoid(m1 - m2) * ones16
    w2b_ref[...] = jax.nn.sigmoid(m2 - m1) * ones16

    oh1 = (lane == i1).astype(jnp.float32)
    oh2 = (lane == i2).astype(jnp.float32)
    ohsum = oh1 + oh2
    pch = 256
    nch = T_TOKENS // pch
    r_io = jax.lax.broadcasted_iota(jnp.int32, (pch, pch), 0)
    c_io = jax.lax.broadcasted_iota(jnp.int32, (pch, pch), 1)
    ltri = (r_io > c_io).astype(jnp.float32)
    oh3 = ohsum.reshape(nch, pch, N_EXPERTS)
    local = jax.lax.dot_general(ltri, oh3, (((1,), (1,)), ((), ())))
    local = local.transpose(1, 0, 2)
    chtot = jnp.sum(oh3, axis=1)
    n_r = jax.lax.broadcasted_iota(jnp.int32, (nch, nch), 0)
    n_c = jax.lax.broadcasted_iota(jnp.int32, (nch, nch), 1)
    ltri_n = (n_r > n_c).astype(jnp.float32)
    choff = jax.lax.dot(ltri_n, chtot)
    excl = (local + choff[:, None, :]).reshape(T_TOKENS, N_EXPERTS)
    rank1 = jnp.sum(oh1 * excl, axis=1, keepdims=True)
    rank2 = jnp.sum(oh2 * (excl + oh1), axis=1, keepdims=True)

    cnt = jnp.sum(ohsum, axis=0, keepdims=True)
    nb_e = jnp.floor((cnt + (BT - 1)) * (1.0 / BT))
    e_r = jax.lax.broadcasted_iota(jnp.int32, (N_EXPERTS, N_EXPERTS), 0)
    e_c = jax.lax.broadcasted_iota(jnp.int32, (N_EXPERTS, N_EXPERTS), 1)
    utri = (e_r <= e_c).astype(jnp.float32)
    nb_csum = jax.lax.dot(nb_e, utri)
    pstart = (nb_csum - nb_e) * float(BT)

    pos1 = jnp.sum(oh1 * pstart, axis=1, keepdims=True) + rank1
    pos2 = jnp.sum(oh2 * pstart, axis=1, keepdims=True) + rank2
    pos1_ref[...] = pos1.astype(jnp.int32)
    pos2_ref[...] = pos2.astype(jnp.int32)

    j_io = jax.lax.broadcasted_iota(
        jnp.int32, (NBLK_PAD, N_EXPERTS), 0).astype(jnp.float32)
    ge = (j_io >= nb_csum).astype(jnp.float32)
    blke = jnp.minimum(jnp.sum(ge, axis=1, keepdims=True),
                       float(N_EXPERTS - 1))
    blke_ref[...] = blke.astype(jnp.int32)


def _shared_body(x_ref, ws1_ref, bs1_ref, ws2_ref, bs2_ref, base_ref):
    x = x_ref[...]
    h = x @ ws1_ref[...] + bs1_ref[...]
    h = h * jax.nn.sigmoid(h)
    base_ref[...] = x + h @ ws2_ref[...] + bs2_ref[...]


def _expert_ffn_body(blk_e_ref, xs_ref, we1_ref, be1_ref, we2_ref, be2_ref,
                     y_ref):
    xg = xs_ref[...]
    h = xg @ we1_ref[0] + be1_ref[0]
    h = h * jax.nn.sigmoid(h)
    y_ref[...] = h @ we2_ref[0] + be2_ref[0]


_NCORES = 2
_NSUB = 16
_NWORK = _NCORES * _NSUB
TPW = T_TOKENS // _NWORK


def _dispatch_sc(xf_hbm, pos1_hbm, pos2_hbm, xs_hbm, idx1_v, idx2_v, xbuf,
                 sem):
    wid = lax.axis_index("s") * _NCORES + lax.axis_index("c")
    start = wid * TPW
    pltpu.sync_copy(pos1_hbm.at[pl.ds(start, TPW)], idx1_v)
    pltpu.sync_copy(pos2_hbm.at[pl.ds(start, TPW)], idx2_v)
    pltpu.sync_copy(xf_hbm.at[pl.ds(start, TPW)], xbuf)
    pltpu.async_copy(xbuf, xs_hbm.at[idx1_v], sem).wait()
    pltpu.async_copy(xbuf, xs_hbm.at[idx2_v], sem).wait()


NCC = 4
CC = TPW // NCC


def _combine_sc(base_hbm, y_hbm, pos1_hbm, pos2_hbm, w1b_hbm, w2b_hbm,
                out_hbm,
                idx1_a, idx2_a, w1_a, w2_a, y1_a, y2_a, ob_a,
                idx1_b, idx2_b, w1_b, w2_b, y1_b, y2_b, ob_b,
                sem, sem_o):
    wid = lax.axis_index("s") * _NCORES + lax.axis_index("c")
    start = wid * TPW
    slots = [(idx1_a, idx2_a, w1_a, w2_a, y1_a, y2_a, ob_a),
             (idx1_b, idx2_b, w1_b, w2_b, y1_b, y2_b, ob_b)]

    def fire(c, slot):
        i1v, i2v, w1v, w2v, y1b, y2b, obb = slot
        cs = start + c * CC
        pltpu.sync_copy(pos1_hbm.at[pl.ds(cs, CC)], i1v)
        pltpu.sync_copy(pos2_hbm.at[pl.ds(cs, CC)], i2v)
        pltpu.sync_copy(w1b_hbm.at[pl.ds(cs, CC)], w1v)
        pltpu.sync_copy(w2b_hbm.at[pl.ds(cs, CC)], w2v)
        return (pltpu.async_copy(y_hbm.at[i1v], y1b, sem),
                pltpu.async_copy(y_hbm.at[i2v], y2b, sem),
                pltpu.async_copy(base_hbm.at[pl.ds(cs, CC)], obb, sem))

    pend = fire(0, slots[0])
    out_pend = [None, None]
    for c in range(NCC):
        slot = slots[c % 2]
        nxt = None
        if c + 1 < NCC:
            nslot = (c + 1) % 2
            if out_pend[nslot] is not None:
                out_pend[nslot].wait()
                out_pend[nslot] = None
            nxt = fire(c + 1, slots[nslot])
        for dsc in pend:
            dsc.wait()
        i1v, i2v, w1v, w2v, y1b, y2b, obb = slot

        def per_token(i, _):
            w1s = w1v[i, pl.ds(0, 16)]
            w2s = w2v[i, pl.ds(0, 16)]

            def per_vec(j, _):
                sl = (i, pl.ds(j * 16, 16))
                obb[sl] = obb[sl] + w1s * y1b[sl] + w2s * y2b[sl]
                return 0

            return lax.fori_loop(0, D_MODEL // 16, per_vec, 0, unroll=16)

        lax.fori_loop(0, CC, per_token, 0)
        out_pend[c % 2] = pltpu.async_copy(
            obb, out_hbm.at[pl.ds(start + c * CC, CC)], sem_o)
        pend = nxt
    for op in out_pend:
        if op is not None:
            op.wait()


def _run_dispatch(xf, pos1, pos2):
    mesh = plsc.VectorSubcoreMesh(core_axis_name="c", subcore_axis_name="s")
    k = functools.partial(
        pl.kernel,
        out_type=jax.ShapeDtypeStruct((NP, D_MODEL), jnp.float32),
        mesh=mesh,
        scratch_types=[
            pltpu.VMEM((TPW,), jnp.int32),
            pltpu.VMEM((TPW,), jnp.int32),
            pltpu.VMEM((TPW, D_MODEL), jnp.float32),
            pltpu.SemaphoreType.DMA,
        ],
    )(_dispatch_sc)
    return k(xf, pos1, pos2)


def _run_combine(base, y, pos1, pos2, w1b, w2b):
    mesh = plsc.VectorSubcoreMesh(core_axis_name="c", subcore_axis_name="s")
    k = functools.partial(
        pl.kernel,
        out_type=jax.ShapeDtypeStruct((T_TOKENS, D_MODEL), jnp.float32),
        mesh=mesh,
        scratch_types=[
            pltpu.VMEM((CC,), jnp.int32),
            pltpu.VMEM((CC,), jnp.int32),
            pltpu.VMEM((CC, 16), jnp.float32),
            pltpu.VMEM((CC, 16), jnp.float32),
            pltpu.VMEM((CC, D_MODEL), jnp.float32),
            pltpu.VMEM((CC, D_MODEL), jnp.float32),
            pltpu.VMEM((CC, D_MODEL), jnp.float32),
            pltpu.VMEM((CC,), jnp.int32),
            pltpu.VMEM((CC,), jnp.int32),
            pltpu.VMEM((CC, 16), jnp.float32),
            pltpu.VMEM((CC, 16), jnp.float32),
            pltpu.VMEM((CC, D_MODEL), jnp.float32),
            pltpu.VMEM((CC, D_MODEL), jnp.float32),
            pltpu.VMEM((CC, D_MODEL), jnp.float32),
            pltpu.SemaphoreType.DMA,
            pltpu.SemaphoreType.DMA,
        ],
    )(_combine_sc)
    return k(base, y, pos1, pos2, w1b, w2b)


def kernel(x, Ws1, bs1, Ws2, bs2, We1, be1, We2, be2, Wg, bg, bias):
    B, T, d = x.shape
    xf = x.reshape(T, d)

    pos1, pos2, blke, w1b, w2b = pl.pallas_call(
        _route_body,
        out_shape=[
            jax.ShapeDtypeStruct((T, 1), jnp.int32),
            jax.ShapeDtypeStruct((T, 1), jnp.int32),
            jax.ShapeDtypeStruct((NBLK_PAD, 1), jnp.int32),
            jax.ShapeDtypeStruct((T, 16), jnp.float32),
            jax.ShapeDtypeStruct((T, 16), jnp.float32),
        ],
        compiler_params=pltpu.CompilerParams(
            vmem_limit_bytes=100 * 1024 * 1024),
    )(xf, Wg, (bg + bias).reshape(1, N_EXPERTS))
    p0 = pos1[:, 0]
    p1 = pos2[:, 0]
    blk_e = blke[:NBLK, 0]

    xs = _run_dispatch(xf, p0, p1)

    tb = 256
    base = pl.pallas_call(
        _shared_body,
        grid=(T // tb,),
        in_specs=[
            pl.BlockSpec((tb, d), lambda b: (b, 0)),
            pl.BlockSpec((d, D_FF), lambda b: (0, 0)),
            pl.BlockSpec((1, D_FF), lambda b: (0, 0)),
            pl.BlockSpec((D_FF, d), lambda b: (0, 0)),
            pl.BlockSpec((1, d), lambda b: (0, 0)),
        ],
        out_specs=pl.BlockSpec((tb, d), lambda b: (b, 0)),
        out_shape=jax.ShapeDtypeStruct((T, d), jnp.float32),
    )(xf, Ws1, bs1.reshape(1, D_FF), Ws2, bs2.reshape(1, d))

    grid_spec = pltpu.PrefetchScalarGridSpec(
        num_scalar_prefetch=1,
        grid=(NBLK,),
        in_specs=[
            pl.BlockSpec((BT, d), lambda b, s: (b, 0)),
            pl.BlockSpec((1, d, D_FF), lambda b, s: (s[b], 0, 0)),
            pl.BlockSpec((1, 1, D_FF), lambda b, s: (s[b], 0, 0)),
            pl.BlockSpec((1, D_FF, d), lambda b, s: (s[b], 0, 0)),
            pl.BlockSpec((1, 1, d), lambda b, s: (s[b], 0, 0)),
        ],
        out_specs=pl.BlockSpec((BT, d), lambda b, s: (b, 0)),
    )
    y = pl.pallas_call(
        _expert_ffn_body,
        grid_spec=grid_spec,
        out_shape=jax.ShapeDtypeStruct((NP, d), jnp.float32),
        compiler_params=pltpu.CompilerParams(
            vmem_limit_bytes=110 * 1024 * 1024),
    )(blk_e, xs, We1, be1.reshape(N_EXPERTS, 1, D_FF), We2,
      be2.reshape(N_EXPERTS, 1, d))

    out = _run_combine(base, y, p0, p1, w1b, w2b)
    return out.reshape(B, T, d)


# speedup vs baseline: 1.0040x; 1.0040x over previous
import functools

import jax
import jax.numpy as jnp
from jax import lax
from jax.experimental import pallas as pl
from jax.experimental.pallas import tpu as pltpu
from jax.experimental.pallas import tpu_sc as plsc

D_MODEL = 768
N_EXPERTS = 64
TOP_K = 2
D_FF = 4 * D_MODEL
T_TOKENS = 2048
N_ASSIGN = T_TOKENS * TOP_K

BT = 128
NBLK = N_ASSIGN // BT + N_EXPERTS - 1
NP = NBLK * BT
NBLK_PAD = 128


def _route_body(x_ref, wg_ref, bgb_ref,
                pos1_ref, pos2_ref, blke_ref, w1b_ref, w2b_ref):
    x = x_ref[...]
    logits = x @ wg_ref[...] + bgb_ref[...]
    s = jax.nn.sigmoid(logits)
    lane = jax.lax.broadcasted_iota(jnp.int32, s.shape, 1)
    big = jnp.int32(N_EXPERTS)
    m1 = jnp.max(s, axis=1, keepdims=True)
    i1 = jnp.min(jnp.where(s == m1, lane, big), axis=1, keepdims=True)
    s2 = jnp.where(lane == i1, -jnp.inf, s)
    m2 = jnp.max(s2, axis=1, keepdims=True)
    i2 = jnp.min(jnp.where(s2 == m2, lane, big), axis=1, keepdims=True)
    ones16 = jnp.ones((1, 16), jnp.float32)
    w1b_ref[...] = jax.nn.sigmoid(m1 - m2) * ones16
    w2b_ref[...] = jax.nn.sigmoid(m2 - m1) * ones16

    oh1 = (lane == i1).astype(jnp.float32)
    oh2 = (lane == i2).astype(jnp.float32)
    ohsum = oh1 + oh2
    pch = 256
    nch = T_TOKENS // pch
    r_io = jax.lax.broadcasted_iota(jnp.int32, (pch, pch), 0)
    c_io = jax.lax.broadcasted_iota(jnp.int32, (pch, pch), 1)
    ltri = (r_io > c_io).astype(jnp.float32)
    oh3 = ohsum.reshape(nch, pch, N_EXPERTS)
    local = jax.lax.dot_general(ltri, oh3, (((1,), (1,)), ((), ())))
    local = local.transpose(1, 0, 2)
    chtot = jnp.sum(oh3, axis=1)
    n_r = jax.lax.broadcasted_iota(jnp.int32, (nch, nch), 0)
    n_c = jax.lax.broadcasted_iota(jnp.int32, (nch, nch), 1)
    ltri_n = (n_r > n_c).astype(jnp.float32)
    choff = jax.lax.dot(ltri_n, chtot)
    excl = (local + choff[:, None, :]).reshape(T_TOKENS, N_EXPERTS)
    rank1 = jnp.sum(oh1 * excl, axis=1, keepdims=True)
    rank2 = jnp.sum(oh2 * (excl + oh1), axis=1, keepdims=True)

    cnt = jnp.sum(ohsum, axis=0, keepdims=True)
    nb_e = jnp.floor((cnt + (BT - 1)) * (1.0 / BT))
    e_r = jax.lax.broadcasted_iota(jnp.int32, (N_EXPERTS, N_EXPERTS), 0)
    e_c = jax.lax.broadcasted_iota(jnp.int32, (N_EXPERTS, N_EXPERTS), 1)
    utri = (e_r <= e_c).astype(jnp.float32)
    nb_csum = jax.lax.dot(nb_e, utri)
    pstart = (nb_csum - nb_e) * float(BT)

    pos1 = jnp.sum(oh1 * pstart, axis=1, keepdims=True) + rank1
    pos2 = jnp.sum(oh2 * pstart, axis=1, keepdims=True) + rank2
    pos1_ref[...] = pos1.astype(jnp.int32)
    pos2_ref[...] = pos2.astype(jnp.int32)

    j_io = jax.lax.broadcasted_iota(
        jnp.int32, (NBLK_PAD, N_EXPERTS), 0).astype(jnp.float32)
    ge = (j_io >= nb_csum).astype(jnp.float32)
    blke = jnp.minimum(jnp.sum(ge, axis=1, keepdims=True),
                       float(N_EXPERTS - 1))
    blke_ref[...] = blke.astype(jnp.int32)


def _shared_body(x_ref, ws1_ref, bs1_ref, ws2_ref, bs2_ref, base_ref):
    x = x_ref[...]
    h = x @ ws1_ref[...] + bs1_ref[...]
    h = h * jax.nn.sigmoid(h)
    base_ref[...] = x + h @ ws2_ref[...] + bs2_ref[...]


def _expert_ffn_body(blk_e_ref, xs_ref, we1_ref, be1_ref, we2_ref, be2_ref,
                     y_ref):
    xg = xs_ref[...]
    h = xg @ we1_ref[0] + be1_ref[0]
    h = h * jax.nn.sigmoid(h)
    y_ref[...] = h @ we2_ref[0] + be2_ref[0]


_NCORES = 2
_NSUB = 16
_NWORK = _NCORES * _NSUB
TPW = T_TOKENS // _NWORK


def _dispatch_sc(xf_hbm, pos1_hbm, pos2_hbm, xs_hbm, idx1_v, idx2_v, xbuf,
                 sem):
    wid = lax.axis_index("s") * _NCORES + lax.axis_index("c")
    start = wid * TPW
    pltpu.sync_copy(pos1_hbm.at[pl.ds(start, TPW)], idx1_v)
    pltpu.sync_copy(pos2_hbm.at[pl.ds(start, TPW)], idx2_v)
    pltpu.sync_copy(xf_hbm.at[pl.ds(start, TPW)], xbuf)
    pltpu.async_copy(xbuf, xs_hbm.at[idx1_v], sem).wait()
    pltpu.async_copy(xbuf, xs_hbm.at[idx2_v], sem).wait()


NCC = 4
CC = TPW // NCC


def _combine_sc(base_hbm, y_hbm, pos1_hbm, pos2_hbm, w1b_hbm, w2b_hbm,
                out_hbm,
                idx1_a, idx2_a, w1_a, w2_a, y1_a, y2_a, ob_a,
                idx1_b, idx2_b, w1_b, w2_b, y1_b, y2_b, ob_b,
                sem, sem_o):
    wid = lax.axis_index("s") * _NCORES + lax.axis_index("c")
    start = wid * TPW
    slots = [(idx1_a, idx2_a, w1_a, w2_a, y1_a, y2_a, ob_a),
             (idx1_b, idx2_b, w1_b, w2_b, y1_b, y2_b, ob_b)]

    def fire(c, slot):
        i1v, i2v, w1v, w2v, y1b, y2b, obb = slot
        cs = start + c * CC
        pltpu.sync_copy(pos1_hbm.at[pl.ds(cs, CC)], i1v)
        pltpu.sync_copy(pos2_hbm.at[pl.ds(cs, CC)], i2v)
        pltpu.sync_copy(w1b_hbm.at[pl.ds(cs, CC)], w1v)
        pltpu.sync_copy(w2b_hbm.at[pl.ds(cs, CC)], w2v)
        return (pltpu.async_copy(y_hbm.at[i1v], y1b, sem),
                pltpu.async_copy(y_hbm.at[i2v], y2b, sem),
                pltpu.async_copy(base_hbm.at[pl.ds(cs, CC)], obb, sem))

    pend = fire(0, slots[0])
    out_pend = [None, None]
    for c in range(NCC):
        slot = slots[c % 2]
        nxt = None
        if c + 1 < NCC:
            nslot = (c + 1) % 2
            if out_pend[nslot] is not None:
                out_pend[nslot].wait()
                out_pend[nslot] = None
            nxt = fire(c + 1, slots[nslot])
        for dsc in pend:
            dsc.wait()
        i1v, i2v, w1v, w2v, y1b, y2b, obb = slot

        def per_token(i, _):
            w1s = w1v[i, pl.ds(0, 16)]
            w2s = w2v[i, pl.ds(0, 16)]

            def per_vec(j, _):
                sl = (i, pl.ds(j * 16, 16))
                obb[sl] = obb[sl] + w1s * y1b[sl] + w2s * y2b[sl]
                return 0

            return lax.fori_loop(0, D_MODEL // 16, per_vec, 0, unroll=8)

        lax.fori_loop(0, CC, per_token, 0)
        out_pend[c % 2] = pltpu.async_copy(
            obb, out_hbm.at[pl.ds(start + c * CC, CC)], sem_o)
        pend = nxt
    for op in out_pend:
        if op is not None:
            op.wait()


def _run_dispatch(xf, pos1, pos2):
    mesh = plsc.VectorSubcoreMesh(core_axis_name="c", subcore_axis_name="s")
    k = functools.partial(
        pl.kernel,
        out_type=jax.ShapeDtypeStruct((NP, D_MODEL), jnp.float32),
        mesh=mesh,
        scratch_types=[
            pltpu.VMEM((TPW,), jnp.int32),
            pltpu.VMEM((TPW,), jnp.int32),
            pltpu.VMEM((TPW, D_MODEL), jnp.float32),
            pltpu.SemaphoreType.DMA,
        ],
    )(_dispatch_sc)
    return k(xf, pos1, pos2)


def _run_combine(base, y, pos1, pos2, w1b, w2b):
    mesh = plsc.VectorSubcoreMesh(core_axis_name="c", subcore_axis_name="s")
    k = functools.partial(
        pl.kernel,
        out_type=jax.ShapeDtypeStruct((T_TOKENS, D_MODEL), jnp.float32),
        mesh=mesh,
        scratch_types=[
            pltpu.VMEM((CC,), jnp.int32),
            pltpu.VMEM((CC,), jnp.int32),
            pltpu.VMEM((CC, 16), jnp.float32),
            pltpu.VMEM((CC, 16), jnp.float32),
            pltpu.VMEM((CC, D_MODEL), jnp.float32),
            pltpu.VMEM((CC, D_MODEL), jnp.float32),
            pltpu.VMEM((CC, D_MODEL), jnp.float32),
            pltpu.VMEM((CC,), jnp.int32),
            pltpu.VMEM((CC,), jnp.int32),
            pltpu.VMEM((CC, 16), jnp.float32),
            pltpu.VMEM((CC, 16), jnp.float32),
            pltpu.VMEM((CC, D_MODEL), jnp.float32),
            pltpu.VMEM((CC, D_MODEL), jnp.float32),
            pltpu.VMEM((CC, D_MODEL), jnp.float32),
            pltpu.SemaphoreType.DMA,
            pltpu.SemaphoreType.DMA,
        ],
    )(_combine_sc)
    return k(base, y, pos1, pos2, w1b, w2b)


def kernel(x, Ws1, bs1, Ws2, bs2, We1, be1, We2, be2, Wg, bg, bias):
    B, T, d = x.shape
    xf = x.reshape(T, d)

    pos1, pos2, blke, w1b, w2b = pl.pallas_call(
        _route_body,
        out_shape=[
            jax.ShapeDtypeStruct((T, 1), jnp.int32),
            jax.ShapeDtypeStruct((T, 1), jnp.int32),
            jax.ShapeDtypeStruct((NBLK_PAD, 1), jnp.int32),
            jax.ShapeDtypeStruct((T, 16), jnp.float32),
            jax.ShapeDtypeStruct((T, 16), jnp.float32),
        ],
        compiler_params=pltpu.CompilerParams(
            vmem_limit_bytes=100 * 1024 * 1024),
    )(xf, Wg, (bg + bias).reshape(1, N_EXPERTS))
    p0 = pos1[:, 0]
    p1 = pos2[:, 0]
    blk_e = blke[:NBLK, 0]

    xs = _run_dispatch(xf, p0, p1)

    tb = 512
    base = pl.pallas_call(
        _shared_body,
        grid=(T // tb,),
        in_specs=[
            pl.BlockSpec((tb, d), lambda b: (b, 0)),
            pl.BlockSpec((d, D_FF), lambda b: (0, 0)),
            pl.BlockSpec((1, D_FF), lambda b: (0, 0)),
            pl.BlockSpec((D_FF, d), lambda b: (0, 0)),
            pl.BlockSpec((1, d), lambda b: (0, 0)),
        ],
        out_specs=pl.BlockSpec((tb, d), lambda b: (b, 0)),
        out_shape=jax.ShapeDtypeStruct((T, d), jnp.float32),
    )(xf, Ws1, bs1.reshape(1, D_FF), Ws2, bs2.reshape(1, d))

    grid_spec = pltpu.PrefetchScalarGridSpec(
        num_scalar_prefetch=1,
        grid=(NBLK,),
        in_specs=[
            pl.BlockSpec((BT, d), lambda b, s: (b, 0)),
            pl.BlockSpec((1, d, D_FF), lambda b, s: (s[b], 0, 0)),
            pl.BlockSpec((1, 1, D_FF), lambda b, s: (s[b], 0, 0)),
            pl.BlockSpec((1, D_FF, d), lambda b, s: (s[b], 0, 0)),
            pl.BlockSpec((1, 1, d), lambda b, s: (s[b], 0, 0)),
        ],
        out_specs=pl.BlockSpec((BT, d), lambda b, s: (b, 0)),
    )
    y = pl.pallas_call(
        _expert_ffn_body,
        grid_spec=grid_spec,
        out_shape=jax.ShapeDtypeStruct((NP, d), jnp.float32),
        compiler_params=pltpu.CompilerParams(
            vmem_limit_bytes=110 * 1024 * 1024),
    )(blk_e, xs, We1, be1.reshape(N_EXPERTS, 1, D_FF), We2,
      be2.reshape(N_EXPERTS, 1, d))

    out = _run_combine(base, y, p0, p1, w1b, w2b)
    return out.reshape(B, T, d)
